# Initial kernel scaffold; baseline (speedup 1.0000x reference)
import jax
import jax.numpy as jnp
from jax.experimental import pallas as pl


def kernel(labels, token_embedding, tokenized_prompts, ctx):
    raise NotImplementedError("write your pallas kernel here")


# speedup vs baseline: 1.2064x; 1.2064x over previous
import functools

import jax
import jax.numpy as jnp
from jax import lax
from jax.experimental import pallas as pl
from jax.experimental.pallas import tpu as pltpu
from jax.experimental.pallas import tpu_sc as plsc


def kernel(labels, token_embedding, tokenized_prompts, ctx):
    B = labels.shape[0]
    C, T = tokenized_prompts.shape
    V, D = token_embedding.shape
    n_ctx = ctx.shape[0]
    n_suf = T - 1 - n_ctx

    info = plsc.get_sparse_core_info()
    NC, NS = info.num_cores, info.num_subcores
    NW = NC * NS
    rows_per_w = B // NW

    G = ((1 + n_suf + 7) // 8) * 8
    W = ((G + n_ctx + 7) // 8) * 8
    toks_tab = jnp.zeros((C, W), jnp.int32)
    toks_tab = toks_tab.at[:, 0].set(tokenized_prompts[:, 0])
    toks_tab = toks_tab.at[:, 1 : 1 + n_suf].set(tokenized_prompts[:, 1 + n_ctx :])

    mesh = plsc.VectorSubcoreMesh(core_axis_name="c", subcore_axis_name="s")

    @functools.partial(
        pl.kernel,
        mesh=mesh,
        out_type=jax.ShapeDtypeStruct((B, T, D), jnp.float32),
        compiler_params=pltpu.CompilerParams(use_tc_tiling_on_sc=False),
        scratch_types=[
            pltpu.VMEM((rows_per_w,), jnp.int32),
            pltpu.VMEM((rows_per_w, W), jnp.int32),
            pltpu.VMEM((2, G + n_ctx, D), jnp.float32),
            pltpu.SemaphoreType.DMA,
            pltpu.SemaphoreType.DMA,
            pltpu.SemaphoreType.DMA,
            pltpu.SemaphoreType.DMA,
        ],
    )
    def _prompt_gather(labels_hbm, emb_hbm, toks_hbm, ctx_hbm, out_hbm,
                       labels_v, toks_v, buf, gsem0, gsem1, wsem0, wsem1):
        wid = lax.axis_index("s") * NC + lax.axis_index("c")
        base = wid * rows_per_w
        gsems = (gsem0, gsem1)
        wsems = (wsem0, wsem1)

        pltpu.sync_copy(labels_hbm.at[pl.ds(base, rows_per_w)], labels_v)
        pltpu.async_copy(toks_hbm.at[labels_v], toks_v, gsem0).wait()

        pltpu.sync_copy(ctx_hbm, buf.at[0, pl.ds(G, n_ctx)])
        pltpu.sync_copy(ctx_hbm, buf.at[1, pl.ds(G, n_ctx)])

        def gather_copy(i, p):
            return pltpu.make_async_copy(
                emb_hbm.at[toks_v.at[i, pl.ds(0, G)]],
                buf.at[p, pl.ds(0, G)], gsems[p])

        def write_copies(i, p):
            return (
                pltpu.make_async_copy(
                    buf.at[p, pl.ds(0, 1)],
                    out_hbm.at[base + i, pl.ds(0, 1)], wsems[p]),
                pltpu.make_async_copy(
                    buf.at[p, pl.ds(G, n_ctx)],
                    out_hbm.at[base + i, pl.ds(1, n_ctx)], wsems[p]),
                pltpu.make_async_copy(
                    buf.at[p, pl.ds(1, n_suf)],
                    out_hbm.at[base + i, pl.ds(1 + n_ctx, n_suf)], wsems[p]),
            )

        def fire_writes(i, p):
            for c in write_copies(i, p):
                c.start()

        def wait_writes(i, p):
            for c in write_copies(i, p):
                c.wait()

        gather_copy(0, 0).start()
        gather_copy(1, 1).start()
        gather_copy(0, 0).wait()
        fire_writes(0, 0)
        gather_copy(1, 1).wait()
        fire_writes(1, 1)

        def body(k, carry):
            for q in range(2):
                i = 2 * k + q
                wait_writes(i - 2, q)
                gather_copy(i, q).start()
                gather_copy(i, q).wait()
                fire_writes(i, q)
            return carry

        lax.fori_loop(1, rows_per_w // 2, body, 0)
        wait_writes(rows_per_w - 2, 0)
        wait_writes(rows_per_w - 1, 1)

    return _prompt_gather(labels, token_embedding, toks_tab, ctx)
